# baseline (speedup 1.0000x reference)
import functools

import jax
import jax.numpy as jnp
from jax import lax
from jax.experimental import pallas as pl
from jax.experimental.pallas import tpu as pltpu
from jax.experimental.pallas import tpu_sc as plsc

N = 10000
D = 128
N_PAD = 10016
CH = 128
N_SC = 2
N_SUB = 16
N_TILES = N_SC * N_SUB
ZERO_ROWS = N_PAD // N_SUB
NBUF = 3
ROW_BLK = 2000


def _make_spmv(n_chunks: int):
  mesh = plsc.VectorSubcoreMesh(core_axis_name="c", subcore_axis_name="s")
  n_groups = n_chunks // NBUF

  @functools.partial(
      pl.kernel,
      out_type=jax.ShapeDtypeStruct((2 * N_PAD, D), jnp.float32),
      mesh=mesh,
      scratch_types=(
          [pltpu.VMEM((2 * NBUF, CH), jnp.int32),
           pltpu.VMEM((2 * NBUF, CH), jnp.int32),
           pltpu.VMEM_SHARED((N_PAD, D), jnp.float32),
           pltpu.SemaphoreType.DMA]
          + [pltpu.VMEM((CH, D), jnp.float32)] * NBUF
          + [pltpu.SemaphoreType.DMA] * (2 * NBUF)
      ),
      compiler_params=pltpu.CompilerParams(use_tc_tiling_on_sc=False),
  )
  def spmv(y_hbm, src_hbm, dst_hbm, out_hbm, srcb, dstb, acc, isem, *rest):
    bufs = rest[:NBUF]
    gsem = rest[NBUF:2 * NBUF]
    ssem = rest[2 * NBUF:3 * NBUF]
    cid = lax.axis_index("c")
    sid = lax.axis_index("s")
    tid = cid * N_SUB + sid
    row0 = tid * n_chunks

    def zrow(r, _):
      for l in range(D // 16):
        bufs[0][r, pl.ds(l * 16, 16)] = jnp.zeros((16,), jnp.float32)
      return 0
    lax.fori_loop(0, CH, zrow, 0)
    zbase = sid * ZERO_ROWS
    nfull = ZERO_ROWS // CH
    for b in range(nfull):
      pltpu.sync_copy(bufs[0], acc.at[pl.ds(zbase + b * CH, CH)])
    rem = ZERO_ROWS - nfull * CH
    if rem:
      pltpu.sync_copy(bufs[0].at[pl.ds(0, rem)],
                      acc.at[pl.ds(zbase + nfull * CH, rem)])
    plsc.subcore_barrier()

    pltpu.sync_copy(src_hbm.at[pl.ds(row0, NBUF)], srcb.at[pl.ds(0, NBUF)])
    pltpu.sync_copy(dst_hbm.at[pl.ds(row0, NBUF)], dstb.at[pl.ds(0, NBUF)])
    if n_groups > 1:
      pltpu.async_copy(src_hbm.at[pl.ds(row0 + NBUF, NBUF)],
                       srcb.at[pl.ds(NBUF, NBUF)], isem)
      pltpu.async_copy(dst_hbm.at[pl.ds(row0 + NBUF, NBUF)],
                       dstb.at[pl.ds(NBUF, NBUF)], isem)
    for b in range(NBUF):
      pltpu.async_copy(y_hbm.at[srcb.at[b]], bufs[b], gsem[b])

    def group(g, _):
      slot = lax.rem(g, 2) * NBUF
      nslot = lax.rem(g + 1, 2) * NBUF
      rnext = row0 + (g + 1) * NBUF
      rnext2 = row0 + (g + 2) * NBUF

      for b in range(NBUF):
        pltpu.make_async_copy(y_hbm.at[srcb.at[slot + b]], bufs[b],
                              gsem[b]).wait()
        pltpu.async_copy(bufs[b], acc.at[dstb.at[slot + b]], ssem[b], add=True)

      @pl.when(g + 1 < n_groups)
      def _advance():
        pltpu.make_async_copy(src_hbm.at[pl.ds(rnext, NBUF)],
                              srcb.at[pl.ds(nslot, NBUF)], isem).wait()
        pltpu.make_async_copy(dst_hbm.at[pl.ds(rnext, NBUF)],
                              dstb.at[pl.ds(nslot, NBUF)], isem).wait()
        for b in range(NBUF):
          pltpu.make_async_copy(bufs[b], acc.at[dstb.at[slot + b]],
                                ssem[b]).wait()

        @pl.when(g + 2 < n_groups)
        def _load_idx():
          pltpu.async_copy(src_hbm.at[pl.ds(rnext2, NBUF)],
                           srcb.at[pl.ds(slot, NBUF)], isem)
          pltpu.async_copy(dst_hbm.at[pl.ds(rnext2, NBUF)],
                           dstb.at[pl.ds(slot, NBUF)], isem)

        for b in range(NBUF):
          pltpu.async_copy(y_hbm.at[srcb.at[nslot + b]], bufs[b], gsem[b])
      return 0
    lax.fori_loop(0, n_groups, group, 0)

    lslot = lax.rem(n_groups - 1, 2) * NBUF
    for b in range(NBUF):
      pltpu.make_async_copy(bufs[b], acc.at[dstb.at[lslot + b]],
                            ssem[b]).wait()

    plsc.subcore_barrier()
    obase = sid * ZERO_ROWS
    pltpu.sync_copy(acc.at[pl.ds(obase, ZERO_ROWS)],
                    out_hbm.at[pl.ds(cid * N_PAD + obase, ZERO_ROWS)])

  return spmv


def _pad_edges(src, dst, n_chunks):
  e_pad = N_TILES * n_chunks * CH
  pad = e_pad - src.shape[0]
  src_p = jnp.concatenate([src, jnp.zeros((pad,), jnp.int32)])
  dst_p = jnp.concatenate([dst, jnp.full((pad,), N, jnp.int32)])
  return (src_p.reshape(N_TILES * n_chunks, CH),
          dst_p.reshape(N_TILES * n_chunks, CH))


_P_SPEC = pl.BlockSpec((2, ROW_BLK, D), lambda i: (0, i, 0))
_H_SPEC = pl.BlockSpec((ROW_BLK, D), lambda i: (i, 0))
_H_SHAPE = jax.ShapeDtypeStruct((N, D), jnp.float32)
_W_SPEC = pl.BlockSpec((D, D), lambda i: (0, 0))


def _mm_body(x_ref, w_ref, y_ref):
  y_ref[...] = jnp.dot(x_ref[...], w_ref[...], preferred_element_type=jnp.float32)


def _matmul(x, w):
  return pl.pallas_call(
      _mm_body,
      grid=(N // ROW_BLK,),
      in_specs=[_H_SPEC, _W_SPEC],
      out_specs=_H_SPEC,
      out_shape=_H_SHAPE,
  )(x, w)


def _relu_mm_body(p_ref, w_ref, h_ref, y_ref):
  h = jnp.maximum(p_ref[0] + p_ref[1], 0.0)
  h_ref[...] = h
  y_ref[...] = jnp.dot(h, w_ref[...], preferred_element_type=jnp.float32)


def _relu_mm(p, w):
  return pl.pallas_call(
      _relu_mm_body,
      grid=(N // ROW_BLK,),
      in_specs=[_P_SPEC, _W_SPEC],
      out_specs=[_H_SPEC, _H_SPEC],
      out_shape=[_H_SHAPE, _H_SHAPE],
  )(p, w)


def _blend_mm_body(p_ref, hprev_ref, a_ref, w_ref, h_ref, y_ref):
  a = a_ref[0, 0]
  t = jnp.maximum(p_ref[0] + p_ref[1], 0.0)
  h = a * t + (1.0 - a) * hprev_ref[...]
  h_ref[...] = h
  y_ref[...] = jnp.dot(h, w_ref[...], preferred_element_type=jnp.float32)


def _blend_mm(p, hprev, a, w):
  return pl.pallas_call(
      _blend_mm_body,
      grid=(N // ROW_BLK,),
      in_specs=[_P_SPEC, _H_SPEC,
                pl.BlockSpec(memory_space=pltpu.SMEM), _W_SPEC],
      out_specs=[_H_SPEC, _H_SPEC],
      out_shape=[_H_SHAPE, _H_SHAPE],
  )(p, hprev, a, w)


def _final_body(p_ref, hprev_ref, x0_ref, a_ref, out_ref):
  a = a_ref[0, 0]
  t = jnp.maximum(p_ref[0] + p_ref[1], 0.0)
  out_ref[...] = a * t + (1.0 - a) * hprev_ref[...] + x0_ref[...]


def _final(p, hprev, x0, a):
  return pl.pallas_call(
      _final_body,
      grid=(N // ROW_BLK,),
      in_specs=[_P_SPEC, _H_SPEC, _H_SPEC,
                pl.BlockSpec(memory_space=pltpu.SMEM)],
      out_specs=_H_SPEC,
      out_shape=_H_SHAPE,
  )(p, hprev, x0, a)


def kernel(x, edge_index, W1, W2, Wd1, Wd2, alphas):
  src = edge_index[0]
  dst = edge_index[1]
  a = jax.nn.sigmoid(alphas)
  a1 = a[0].reshape(1, 1)
  a2 = a[1].reshape(1, 1)

  sp1, dp1 = _pad_edges(src, dst, 81)
  sp3, dp3 = _pad_edges(src[::2], dst[::2], 42)
  sp4, dp4 = _pad_edges(src[::4], dst[::4], 21)

  spmv81 = _make_spmv(81)
  spmv42 = _make_spmv(42)
  spmv21 = _make_spmv(21)

  y1 = _matmul(x, W1)
  p1 = spmv81(y1, sp1, dp1).reshape(2, N_PAD, D)
  h1, y2 = _relu_mm(p1, W2)
  del h1
  p2 = spmv81(y2, sp1, dp1).reshape(2, N_PAD, D)
  h2, y3 = _relu_mm(p2, Wd1)
  p3 = spmv42(y3, sp3, dp3).reshape(2, N_PAD, D)
  h3, y4 = _blend_mm(p3, h2, a1, Wd2)
  p4 = spmv21(y4, sp4, dp4).reshape(2, N_PAD, D)
  return _final(p4, h3, h2, a2)


# speedup vs baseline: 2.7381x; 2.7381x over previous
import functools

import jax
import jax.numpy as jnp
import numpy as np
from jax import lax
from jax.experimental import pallas as pl
from jax.experimental.pallas import tpu as pltpu
from jax.experimental.pallas import tpu_sc as plsc

N = 10000
D = 128
DH = D // 2
N_PAD = 10240
CH = 128
N_SC = 2
N_SUB = 16
N_TILES = N_SC * N_SUB
ZERO_ROWS = N_PAD // N_SUB
NBUF = 3
ROW_BLK = 2000

PHI = np.concatenate([
    np.stack([np.arange(16) + 32 * t, np.arange(16) + 32 * t + 16],
             axis=1).reshape(-1)
    for t in range(D // 32)
])


def _make_spmv(n_chunks: int):
  mesh = plsc.VectorSubcoreMesh(core_axis_name="c", subcore_axis_name="s")
  n_groups = n_chunks // NBUF

  @functools.partial(
      pl.kernel,
      out_type=jax.ShapeDtypeStruct((2 * N_PAD, DH), jnp.float32),
      mesh=mesh,
      scratch_types=(
          [pltpu.VMEM((n_chunks, CH), jnp.int32),
           pltpu.VMEM((n_chunks, CH), jnp.int32),
           pltpu.VMEM_SHARED((N_PAD, DH), jnp.float32)]
          + [pltpu.VMEM((CH, DH), jnp.bfloat16)] * NBUF
          + [pltpu.VMEM((CH, DH), jnp.float32)] * NBUF
          + [pltpu.SemaphoreType.DMA] * (2 * NBUF)
      ),
      compiler_params=pltpu.CompilerParams(use_tc_tiling_on_sc=False,
                                           needs_layout_passes=False),
  )
  def spmv(y_hbm, src_hbm, dst_hbm, out_hbm, src_v, dst_v, acc, *rest):
    gbufs = rest[:NBUF]
    sbufs = rest[NBUF:2 * NBUF]
    gsem = rest[2 * NBUF:3 * NBUF]
    ssem = rest[3 * NBUF:4 * NBUF]
    cid = lax.axis_index("c")
    sid = lax.axis_index("s")
    tid = cid * N_SUB + sid

    pltpu.sync_copy(src_hbm.at[tid], src_v)
    pltpu.sync_copy(dst_hbm.at[tid], dst_v)

    def zrow(r, _):
      for l in range(DH // 16):
        sbufs[0][r, pl.ds(l * 16, 16)] = jnp.zeros((16,), jnp.float32)
      return 0
    lax.fori_loop(0, CH, zrow, 0)
    zbase = sid * ZERO_ROWS
    for b in range(ZERO_ROWS // CH):
      pltpu.sync_copy(sbufs[0], acc.at[pl.ds(zbase + b * CH, CH)])
    plsc.subcore_barrier()

    def upcast(gb, sb):
      def row(r, _):
        for t in range(DH // 32):
          v = gb[r, pl.ds(32 * t, 32)]
          a, b2 = plsc.unpack(v, format=plsc.PackFormat.INTERLEAVED)
          sb[r, pl.ds(32 * t, 16)] = a
          sb[r, pl.ds(32 * t + 16, 16)] = b2
        return 0
      lax.fori_loop(0, CH, row, 0)

    for b in range(NBUF):
      pltpu.async_copy(y_hbm.at[src_v.at[b]], gbufs[b], gsem[b])

    def group(g, _):
      j0 = g * NBUF

      @pl.when(g > 0)
      def _drain():
        for b in range(NBUF):
          pltpu.make_async_copy(sbufs[b], acc.at[dst_v.at[j0 - NBUF + b]],
                                ssem[b]).wait()

      for b in range(NBUF):
        pltpu.make_async_copy(y_hbm.at[src_v.at[j0 + b]], gbufs[b],
                              gsem[b]).wait()
        upcast(gbufs[b], sbufs[b])
        pltpu.async_copy(sbufs[b], acc.at[dst_v.at[j0 + b]], ssem[b], add=True)

        @pl.when(g + 1 < n_groups)
        def _refill():
          pltpu.async_copy(y_hbm.at[src_v.at[j0 + NBUF + b]], gbufs[b],
                           gsem[b])
      return 0
    lax.fori_loop(0, n_groups, group, 0)

    jl = (n_groups - 1) * NBUF
    for b in range(NBUF):
      pltpu.make_async_copy(sbufs[b], acc.at[dst_v.at[jl + b]], ssem[b]).wait()

    plsc.subcore_barrier()
    obase = sid * ZERO_ROWS
    pltpu.sync_copy(acc.at[pl.ds(obase, ZERO_ROWS)],
                    out_hbm.at[pl.ds(cid * N_PAD + obase, ZERO_ROWS)])

  return spmv


def _pad_edges(src, dst, n_chunks):
  e_pad = N_SUB * n_chunks * CH
  pad = e_pad - src.shape[0]
  src_p = jnp.concatenate([src, jnp.zeros((pad,), jnp.int32)])
  dst_p = jnp.concatenate([dst, jnp.full((pad,), N, jnp.int32)])
  src_p = src_p.reshape(N_SUB, n_chunks, CH)
  dst_p = dst_p.reshape(N_SUB, n_chunks, CH)
  src_both = jnp.concatenate([src_p, src_p + N], axis=0)
  dst_both = jnp.concatenate([dst_p, dst_p], axis=0)
  return src_both, dst_both


def _split(y_ref, res):
  y_ref[0] = res[:, :DH].astype(jnp.bfloat16)
  y_ref[1] = res[:, DH:].astype(jnp.bfloat16)


_Y_SHAPE = jax.ShapeDtypeStruct((2, N, DH), jnp.bfloat16)
_Y_SPEC = pl.BlockSpec((2, ROW_BLK, DH), lambda i: (0, i, 0))
_P_SPEC = pl.BlockSpec((2, ROW_BLK, DH), lambda i: (0, i, 0))
_H_SPEC = pl.BlockSpec((ROW_BLK, D), lambda i: (i, 0))
_H_SHAPE = jax.ShapeDtypeStruct((N, D), jnp.float32)
_W_SPEC = pl.BlockSpec((D, D), lambda i: (0, 0))


def _mm_body(x_ref, w_ref, y_ref):
  _split(y_ref, jnp.dot(x_ref[...], w_ref[...],
                        preferred_element_type=jnp.float32))


def _matmul(x, w):
  return pl.pallas_call(
      _mm_body,
      grid=(N // ROW_BLK,),
      in_specs=[_H_SPEC, _W_SPEC],
      out_specs=_Y_SPEC,
      out_shape=_Y_SHAPE,
  )(x, w)


def _relu_mm_body(p_ref, w_ref, h_ref, y_ref):
  h = jnp.maximum(jnp.concatenate([p_ref[0], p_ref[1]], axis=-1), 0.0)
  h_ref[...] = h
  _split(y_ref, jnp.dot(h, w_ref[...], preferred_element_type=jnp.float32))


def _relu_mm(p, w):
  return pl.pallas_call(
      _relu_mm_body,
      grid=(N // ROW_BLK,),
      in_specs=[_P_SPEC, _W_SPEC],
      out_specs=[_H_SPEC, _Y_SPEC],
      out_shape=[_H_SHAPE, _Y_SHAPE],
  )(p, w)


def _blend_mm_body(p_ref, hprev_ref, a_ref, w_ref, h_ref, y_ref):
  a = a_ref[0, 0]
  t = jnp.maximum(jnp.concatenate([p_ref[0], p_ref[1]], axis=-1), 0.0)
  h = a * t + (1.0 - a) * hprev_ref[...]
  h_ref[...] = h
  _split(y_ref, jnp.dot(h, w_ref[...], preferred_element_type=jnp.float32))


def _blend_mm(p, hprev, a, w):
  return pl.pallas_call(
      _blend_mm_body,
      grid=(N // ROW_BLK,),
      in_specs=[_P_SPEC, _H_SPEC,
                pl.BlockSpec(memory_space=pltpu.SMEM), _W_SPEC],
      out_specs=[_H_SPEC, _Y_SPEC],
      out_shape=[_H_SHAPE, _Y_SHAPE],
  )(p, hprev, a, w)


def _final_body(p_ref, hprev_ref, x0_ref, a_ref, out_ref):
  a = a_ref[0, 0]
  t = jnp.maximum(jnp.concatenate([p_ref[0], p_ref[1]], axis=-1), 0.0)
  out_ref[...] = a * t + (1.0 - a) * hprev_ref[...] + x0_ref[...]


def _final(p, hprev, x0, a):
  return pl.pallas_call(
      _final_body,
      grid=(N // ROW_BLK,),
      in_specs=[_P_SPEC, _H_SPEC, _H_SPEC,
                pl.BlockSpec(memory_space=pltpu.SMEM)],
      out_specs=_H_SPEC,
      out_shape=_H_SHAPE,
  )(p, hprev, x0, a)


def kernel(x, edge_index, W1, W2, Wd1, Wd2, alphas):
  src = edge_index[0]
  dst = edge_index[1]
  a = jax.nn.sigmoid(alphas)
  a1 = a[0].reshape(1, 1)
  a2 = a[1].reshape(1, 1)
  W1p, W2p, Wd1p, Wd2p = (w[:, PHI] for w in (W1, W2, Wd1, Wd2))

  sp1, dp1 = _pad_edges(src, dst, 159)
  sp3, dp3 = _pad_edges(src[::2], dst[::2], 81)
  sp4, dp4 = _pad_edges(src[::4], dst[::4], 42)

  spmv159 = _make_spmv(159)
  spmv81 = _make_spmv(81)
  spmv42 = _make_spmv(42)

  y1 = _matmul(x, W1p).reshape(2 * N, DH)
  p1 = spmv159(y1, sp1, dp1).reshape(2, N_PAD, DH)
  h1, y2 = _relu_mm(p1, W2p)
  del h1
  p2 = spmv159(y2.reshape(2 * N, DH), sp1, dp1).reshape(2, N_PAD, DH)
  h2, y3 = _relu_mm(p2, Wd1p)
  p3 = spmv81(y3.reshape(2 * N, DH), sp3, dp3).reshape(2, N_PAD, DH)
  h3, y4 = _blend_mm(p3, h2, a1, Wd2p)
  p4 = spmv42(y4.reshape(2 * N, DH), sp4, dp4).reshape(2, N_PAD, DH)
  return _final(p4, h3, h2, a2)


# speedup vs baseline: 3.7140x; 1.3564x over previous
import functools

import jax
import jax.numpy as jnp
from jax import lax
from jax.experimental import pallas as pl
from jax.experimental.pallas import tpu as pltpu
from jax.experimental.pallas import tpu_sc as plsc

N = 10000
D = 128
DH = D // 2
N_PAD = 10240
CH = 128
N_SC = 2
N_SUB = 16
N_TILES = N_SC * N_SUB
ZERO_ROWS = N_PAD // N_SUB
NBUF = 8
ROW_BLK = 2000


def _make_spmv(n_chunks: int):
  mesh = plsc.VectorSubcoreMesh(core_axis_name="c", subcore_axis_name="s")
  n_groups = n_chunks // NBUF

  @functools.partial(
      pl.kernel,
      out_type=jax.ShapeDtypeStruct((2 * N_PAD, DH), jnp.bfloat16),
      mesh=mesh,
      scratch_types=(
          [pltpu.VMEM((n_chunks, CH), jnp.int32),
           pltpu.VMEM((n_chunks, CH), jnp.int32),
           pltpu.VMEM_SHARED((N_PAD, DH), jnp.bfloat16)]
          + [pltpu.VMEM((CH, DH), jnp.bfloat16)] * NBUF
          + [pltpu.SemaphoreType.DMA] * (2 * NBUF)
      ),
      compiler_params=pltpu.CompilerParams(use_tc_tiling_on_sc=False,
                                           needs_layout_passes=False),
  )
  def spmv(y_hbm, src_hbm, dst_hbm, out_hbm, src_v, dst_v, acc, *rest):
    bufs = rest[:NBUF]
    gsem = rest[NBUF:2 * NBUF]
    ssem = rest[2 * NBUF:3 * NBUF]
    cid = lax.axis_index("c")
    sid = lax.axis_index("s")
    tid = cid * N_SUB + sid

    pltpu.sync_copy(src_hbm.at[tid], src_v)
    pltpu.sync_copy(dst_hbm.at[tid], dst_v)

    def zrow(r, _):
      for l in range(DH // 32):
        bufs[0][r, pl.ds(l * 32, 32)] = jnp.zeros((32,), jnp.bfloat16)
      return 0
    lax.fori_loop(0, CH, zrow, 0)
    zbase = sid * ZERO_ROWS
    for b in range(ZERO_ROWS // CH):
      pltpu.sync_copy(bufs[0], acc.at[pl.ds(zbase + b * CH, CH)])
    plsc.subcore_barrier()

    for b in range(NBUF):
      pltpu.async_copy(y_hbm.at[src_v.at[b]], bufs[b], gsem[b])

    def group(g, _):
      j0 = g * NBUF
      for b in range(NBUF):
        pltpu.make_async_copy(y_hbm.at[src_v.at[j0 + b]], bufs[b],
                              gsem[b]).wait()
        pltpu.async_copy(bufs[b], acc.at[dst_v.at[j0 + b]], ssem[b], add=True)

      @pl.when(g + 1 < n_groups)
      def _prefetch():
        for b in range(NBUF):
          pltpu.make_async_copy(bufs[b], acc.at[dst_v.at[j0 + b]],
                                ssem[b]).wait()
          pltpu.async_copy(y_hbm.at[src_v.at[j0 + NBUF + b]], bufs[b], gsem[b])
      return 0
    lax.fori_loop(0, n_groups, group, 0)

    jl = (n_groups - 1) * NBUF
    for b in range(NBUF):
      pltpu.make_async_copy(bufs[b], acc.at[dst_v.at[jl + b]], ssem[b]).wait()

    plsc.subcore_barrier()
    obase = sid * ZERO_ROWS
    pltpu.sync_copy(acc.at[pl.ds(obase, ZERO_ROWS)],
                    out_hbm.at[pl.ds(cid * N_PAD + obase, ZERO_ROWS)])

  return spmv


def _pad_edges(src, dst, n_chunks):
  e_pad = N_SUB * n_chunks * CH
  pad = e_pad - src.shape[0]
  src_p = jnp.concatenate([src, jnp.zeros((pad,), jnp.int32)])
  dst_p = jnp.concatenate([dst, jnp.full((pad,), N, jnp.int32)])
  src_p = src_p.reshape(N_SUB, n_chunks, CH)
  dst_p = dst_p.reshape(N_SUB, n_chunks, CH)
  src_both = jnp.concatenate([src_p, src_p + N], axis=0)
  dst_both = jnp.concatenate([dst_p, dst_p], axis=0)
  return src_both, dst_both


def _split(y_ref, res):
  y_ref[0] = res[:, :DH].astype(jnp.bfloat16)
  y_ref[1] = res[:, DH:].astype(jnp.bfloat16)


def _agg(p_ref):
  return jnp.concatenate([p_ref[0], p_ref[1]], axis=-1).astype(jnp.float32)


_Y_SHAPE = jax.ShapeDtypeStruct((2, N, DH), jnp.bfloat16)
_Y_SPEC = pl.BlockSpec((2, ROW_BLK, DH), lambda i: (0, i, 0))
_P_SPEC = pl.BlockSpec((2, ROW_BLK, DH), lambda i: (0, i, 0))
_H_SPEC = pl.BlockSpec((ROW_BLK, D), lambda i: (i, 0))
_H_SHAPE = jax.ShapeDtypeStruct((N, D), jnp.float32)
_W_SPEC = pl.BlockSpec((D, D), lambda i: (0, 0))


def _mm_body(x_ref, w_ref, y_ref):
  _split(y_ref, jnp.dot(x_ref[...], w_ref[...],
                        preferred_element_type=jnp.float32))


def _matmul(x, w):
  return pl.pallas_call(
      _mm_body,
      grid=(N // ROW_BLK,),
      in_specs=[_H_SPEC, _W_SPEC],
      out_specs=_Y_SPEC,
      out_shape=_Y_SHAPE,
  )(x, w)


def _relu_mm_body(p_ref, w_ref, h_ref, y_ref):
  h = jnp.maximum(_agg(p_ref), 0.0)
  h_ref[...] = h
  _split(y_ref, jnp.dot(h, w_ref[...], preferred_element_type=jnp.float32))


def _relu_mm(p, w):
  return pl.pallas_call(
      _relu_mm_body,
      grid=(N // ROW_BLK,),
      in_specs=[_P_SPEC, _W_SPEC],
      out_specs=[_H_SPEC, _Y_SPEC],
      out_shape=[_H_SHAPE, _Y_SHAPE],
  )(p, w)


def _blend_mm_body(p_ref, hprev_ref, a_ref, w_ref, h_ref, y_ref):
  a = a_ref[0, 0]
  t = jnp.maximum(_agg(p_ref), 0.0)
  h = a * t + (1.0 - a) * hprev_ref[...]
  h_ref[...] = h
  _split(y_ref, jnp.dot(h, w_ref[...], preferred_element_type=jnp.float32))


def _blend_mm(p, hprev, a, w):
  return pl.pallas_call(
      _blend_mm_body,
      grid=(N // ROW_BLK,),
      in_specs=[_P_SPEC, _H_SPEC,
                pl.BlockSpec(memory_space=pltpu.SMEM), _W_SPEC],
      out_specs=[_H_SPEC, _Y_SPEC],
      out_shape=[_H_SHAPE, _Y_SHAPE],
  )(p, hprev, a, w)


def _final_body(p_ref, hprev_ref, x0_ref, a_ref, out_ref):
  a = a_ref[0, 0]
  t = jnp.maximum(_agg(p_ref), 0.0)
  out_ref[...] = a * t + (1.0 - a) * hprev_ref[...] + x0_ref[...]


def _final(p, hprev, x0, a):
  return pl.pallas_call(
      _final_body,
      grid=(N // ROW_BLK,),
      in_specs=[_P_SPEC, _H_SPEC, _H_SPEC,
                pl.BlockSpec(memory_space=pltpu.SMEM)],
      out_specs=_H_SPEC,
      out_shape=_H_SHAPE,
  )(p, hprev, x0, a)


def kernel(x, edge_index, W1, W2, Wd1, Wd2, alphas):
  src = edge_index[0]
  dst = edge_index[1]
  a = jax.nn.sigmoid(alphas)
  a1 = a[0].reshape(1, 1)
  a2 = a[1].reshape(1, 1)

  sp1, dp1 = _pad_edges(src, dst, 160)
  sp3, dp3 = _pad_edges(src[::2], dst[::2], 80)
  sp4, dp4 = _pad_edges(src[::4], dst[::4], 40)

  spmv160 = _make_spmv(160)
  spmv80 = _make_spmv(80)
  spmv40 = _make_spmv(40)

  y1 = _matmul(x, W1).reshape(2 * N, DH)
  p1 = spmv160(y1, sp1, dp1).reshape(2, N_PAD, DH)
  h1, y2 = _relu_mm(p1, W2)
  del h1
  p2 = spmv160(y2.reshape(2 * N, DH), sp1, dp1).reshape(2, N_PAD, DH)
  h2, y3 = _relu_mm(p2, Wd1)
  p3 = spmv80(y3.reshape(2 * N, DH), sp3, dp3).reshape(2, N_PAD, DH)
  h3, y4 = _blend_mm(p3, h2, a1, Wd2)
  p4 = spmv40(y4.reshape(2 * N, DH), sp4, dp4).reshape(2, N_PAD, DH)
  return _final(p4, h3, h2, a2)


# speedup vs baseline: 3.7371x; 1.0062x over previous
import functools

import jax
import jax.numpy as jnp
from jax import lax
from jax.experimental import pallas as pl
from jax.experimental.pallas import tpu as pltpu
from jax.experimental.pallas import tpu_sc as plsc

N = 10000
D = 128
DH = D // 2
N_PAD = 10240
CH = 128
N_SC = 2
N_SUB = 16
N_TILES = N_SC * N_SUB
ZERO_ROWS = N_PAD // N_SUB
ROW_BLK = 2000


def _make_spmv(n_chunks: int, NBUF: int):
  mesh = plsc.VectorSubcoreMesh(core_axis_name="c", subcore_axis_name="s")
  n_groups = n_chunks // NBUF

  @functools.partial(
      pl.kernel,
      out_type=jax.ShapeDtypeStruct((2 * N_PAD, DH), jnp.bfloat16),
      mesh=mesh,
      scratch_types=(
          [pltpu.VMEM((n_chunks, CH), jnp.int32),
           pltpu.VMEM((n_chunks, CH), jnp.int32),
           pltpu.VMEM_SHARED((N_PAD, DH), jnp.bfloat16)]
          + [pltpu.VMEM((CH, DH), jnp.bfloat16)] * NBUF
          + [pltpu.SemaphoreType.DMA] * (2 * NBUF)
      ),
      compiler_params=pltpu.CompilerParams(use_tc_tiling_on_sc=False,
                                           needs_layout_passes=False),
  )
  def spmv(y_hbm, src_hbm, dst_hbm, out_hbm, src_v, dst_v, acc, *rest):
    bufs = rest[:NBUF]
    gsem = rest[NBUF:2 * NBUF]
    ssem = rest[2 * NBUF:3 * NBUF]
    cid = lax.axis_index("c")
    sid = lax.axis_index("s")
    tid = cid * N_SUB + sid

    pltpu.sync_copy(src_hbm.at[tid], src_v)
    pltpu.sync_copy(dst_hbm.at[tid], dst_v)
    for b in range(1, NBUF):
      pltpu.async_copy(y_hbm.at[src_v.at[b]], bufs[b], gsem[b])

    def zrow(r, _):
      for l in range(DH // 32):
        bufs[0][r, pl.ds(l * 32, 32)] = jnp.zeros((32,), jnp.bfloat16)
      return 0
    lax.fori_loop(0, CH, zrow, 0)
    zbase = sid * ZERO_ROWS
    for b in range(ZERO_ROWS // CH):
      pltpu.sync_copy(bufs[0], acc.at[pl.ds(zbase + b * CH, CH)])
    pltpu.async_copy(y_hbm.at[src_v.at[0]], bufs[0], gsem[0])
    plsc.subcore_barrier()

    def group(g, _):
      j0 = g * NBUF
      for b in range(NBUF):
        pltpu.make_async_copy(y_hbm.at[src_v.at[j0 + b]], bufs[b],
                              gsem[b]).wait()
        pltpu.async_copy(bufs[b], acc.at[dst_v.at[j0 + b]], ssem[b], add=True)

      @pl.when(g + 1 < n_groups)
      def _prefetch():
        for b in range(NBUF):
          pltpu.make_async_copy(bufs[b], acc.at[dst_v.at[j0 + b]],
                                ssem[b]).wait()
          pltpu.async_copy(y_hbm.at[src_v.at[j0 + NBUF + b]], bufs[b], gsem[b])
      return 0
    lax.fori_loop(0, n_groups, group, 0)

    jl = (n_groups - 1) * NBUF
    for b in range(NBUF):
      pltpu.make_async_copy(bufs[b], acc.at[dst_v.at[jl + b]], ssem[b]).wait()

    plsc.subcore_barrier()
    obase = sid * ZERO_ROWS
    pltpu.sync_copy(acc.at[pl.ds(obase, ZERO_ROWS)],
                    out_hbm.at[pl.ds(cid * N_PAD + obase, ZERO_ROWS)])

  return spmv


def _pad_edges(src, dst, n_chunks):
  e_pad = N_SUB * n_chunks * CH
  pad = e_pad - src.shape[0]
  src_p = jnp.concatenate([src, jnp.zeros((pad,), jnp.int32)])
  dst_p = jnp.concatenate([dst, jnp.full((pad,), N, jnp.int32)])
  src_p = src_p.reshape(N_SUB, n_chunks, CH)
  dst_p = dst_p.reshape(N_SUB, n_chunks, CH)
  src_both = jnp.concatenate([src_p, src_p + N], axis=0)
  dst_both = jnp.concatenate([dst_p, dst_p], axis=0)
  return src_both, dst_both


def _split(y_ref, res):
  y_ref[0] = res[:, :DH].astype(jnp.bfloat16)
  y_ref[1] = res[:, DH:].astype(jnp.bfloat16)


def _agg(p_ref):
  return jnp.concatenate([p_ref[0], p_ref[1]], axis=-1).astype(jnp.float32)


_Y_SHAPE = jax.ShapeDtypeStruct((2, N, DH), jnp.bfloat16)
_Y_SPEC = pl.BlockSpec((2, ROW_BLK, DH), lambda i: (0, i, 0))
_P_SPEC = pl.BlockSpec((2, ROW_BLK, DH), lambda i: (0, i, 0))
_H_SPEC = pl.BlockSpec((ROW_BLK, D), lambda i: (i, 0))
_H_SHAPE = jax.ShapeDtypeStruct((N, D), jnp.float32)
_W_SPEC = pl.BlockSpec((D, D), lambda i: (0, 0))


def _mm_body(x_ref, w_ref, y_ref):
  _split(y_ref, jnp.dot(x_ref[...], w_ref[...],
                        preferred_element_type=jnp.float32))


def _matmul(x, w):
  return pl.pallas_call(
      _mm_body,
      grid=(N // ROW_BLK,),
      in_specs=[_H_SPEC, _W_SPEC],
      out_specs=_Y_SPEC,
      out_shape=_Y_SHAPE,
  )(x, w)


def _relu_mm_body(p_ref, w_ref, h_ref, y_ref):
  h = jnp.maximum(_agg(p_ref), 0.0)
  h_ref[...] = h
  _split(y_ref, jnp.dot(h, w_ref[...], preferred_element_type=jnp.float32))


def _relu_mm(p, w):
  return pl.pallas_call(
      _relu_mm_body,
      grid=(N // ROW_BLK,),
      in_specs=[_P_SPEC, _W_SPEC],
      out_specs=[_H_SPEC, _Y_SPEC],
      out_shape=[_H_SHAPE, _Y_SHAPE],
  )(p, w)


def _blend_mm_body(p_ref, hprev_ref, a_ref, w_ref, h_ref, y_ref):
  a = a_ref[0, 0]
  t = jnp.maximum(_agg(p_ref), 0.0)
  h = a * t + (1.0 - a) * hprev_ref[...]
  h_ref[...] = h
  _split(y_ref, jnp.dot(h, w_ref[...], preferred_element_type=jnp.float32))


def _blend_mm(p, hprev, a, w):
  return pl.pallas_call(
      _blend_mm_body,
      grid=(N // ROW_BLK,),
      in_specs=[_P_SPEC, _H_SPEC,
                pl.BlockSpec(memory_space=pltpu.SMEM), _W_SPEC],
      out_specs=[_H_SPEC, _Y_SPEC],
      out_shape=[_H_SHAPE, _Y_SHAPE],
  )(p, hprev, a, w)


def _final_body(p_ref, hprev_ref, x0_ref, a_ref, out_ref):
  a = a_ref[0, 0]
  t = jnp.maximum(_agg(p_ref), 0.0)
  out_ref[...] = a * t + (1.0 - a) * hprev_ref[...] + x0_ref[...]


def _final(p, hprev, x0, a):
  return pl.pallas_call(
      _final_body,
      grid=(N // ROW_BLK,),
      in_specs=[_P_SPEC, _H_SPEC, _H_SPEC,
                pl.BlockSpec(memory_space=pltpu.SMEM)],
      out_specs=_H_SPEC,
      out_shape=_H_SHAPE,
  )(p, hprev, x0, a)


def kernel(x, edge_index, W1, W2, Wd1, Wd2, alphas):
  src = edge_index[0]
  dst = edge_index[1]
  a = jax.nn.sigmoid(alphas)
  a1 = a[0].reshape(1, 1)
  a2 = a[1].reshape(1, 1)

  sp1, dp1 = _pad_edges(src, dst, 160)
  sp3, dp3 = _pad_edges(src[::2], dst[::2], 80)
  sp4, dp4 = _pad_edges(src[::4], dst[::4], 40)

  spmv160 = _make_spmv(160, 8)
  spmv80 = _make_spmv(80, 8)
  spmv40 = _make_spmv(40, 8)

  y1 = _matmul(x, W1).reshape(2 * N, DH)
  p1 = spmv160(y1, sp1, dp1).reshape(2, N_PAD, DH)
  h1, y2 = _relu_mm(p1, W2)
  del h1
  p2 = spmv160(y2.reshape(2 * N, DH), sp1, dp1).reshape(2, N_PAD, DH)
  h2, y3 = _relu_mm(p2, Wd1)
  p3 = spmv80(y3.reshape(2 * N, DH), sp3, dp3).reshape(2, N_PAD, DH)
  h3, y4 = _blend_mm(p3, h2, a1, Wd2)
  p4 = spmv40(y4.reshape(2 * N, DH), sp4, dp4).reshape(2, N_PAD, DH)
  return _final(p4, h3, h2, a2)
